# baseline (speedup 1.0000x reference)
import functools

import jax
import jax.numpy as jnp
from jax import lax
from jax.experimental import pallas as pl
from jax.experimental.pallas import tpu as pltpu
from jax.experimental.pallas import tpu_sc as plsc

_CHUNK = 128
_G = 1
_PAD_BLK = 8000


def _pad_block(src_ref, dst_ref):
    dst_ref[:, : src_ref.shape[1]] = src_ref[...]
    dst_ref[:, src_ref.shape[1] :] = jnp.zeros(
        (src_ref.shape[0], dst_ref.shape[1] - src_ref.shape[1]), src_ref.dtype
    )


@functools.cache
def _build_pad(vocab, emb_dim):
    return pl.pallas_call(
        _pad_block,
        grid=(vocab // _PAD_BLK,),
        in_specs=[pl.BlockSpec((_PAD_BLK, emb_dim), lambda i: (i, 0))],
        out_specs=pl.BlockSpec((_PAD_BLK, 2 * emb_dim), lambda i: (i, 0)),
        out_shape=jax.ShapeDtypeStruct((vocab, 2 * emb_dim), jnp.float32),
    )


@functools.cache
def _build_gather(total, emb_dim, n_workers, num_cores):
    per_worker = total // n_workers
    chunks = per_worker // _CHUNK
    steps = chunks // _G
    rows_per_step = _G * _CHUNK
    mesh = plsc.VectorSubcoreMesh(core_axis_name="c", subcore_axis_name="s")

    @functools.partial(
        pl.kernel,
        mesh=mesh,
        out_type=jax.ShapeDtypeStruct((total, emb_dim), jnp.float32),
        scratch_types=[
            pltpu.VMEM((chunks, _CHUNK), jnp.int32),
            pltpu.VMEM((rows_per_step, 2 * emb_dim), jnp.float32),
            pltpu.VMEM((rows_per_step, 2 * emb_dim), jnp.float32),
            pltpu.VMEM((rows_per_step, emb_dim), jnp.float32),
            pltpu.VMEM((rows_per_step, emb_dim), jnp.float32),
            pltpu.SemaphoreType.DMA,
            pltpu.SemaphoreType.DMA,
            pltpu.SemaphoreType.DMA,
            pltpu.SemaphoreType.DMA,
        ],
        compiler_params=pltpu.CompilerParams(
            use_tc_tiling_on_sc=True, needs_layout_passes=False
        ),
    )
    def gather_kernel(
        idx_hbm, table_hbm, out_hbm,
        idx_v, buf0, buf1, packed0, packed1, sem0, sem1, wsem0, wsem1,
    ):
        wid = lax.axis_index("s") * num_cores + lax.axis_index("c")
        pltpu.sync_copy(idx_hbm.at[wid], idx_v)
        out_base = wid * per_worker
        bufs = (buf0, buf1)
        sems = (sem0, sem1)
        packs = (packed0, packed1)
        wsems = (wsem0, wsem1)

        def fire(t, buf, sem):
            for g in range(_G):
                pltpu.async_copy(
                    table_hbm.at[idx_v.at[t * _G + g]],
                    buf.at[pl.ds(g * _CHUNK, _CHUNK)],
                    sem,
                )

        def drain(buf, sem):
            pltpu.make_async_copy(
                table_hbm.at[pl.ds(0, rows_per_step)], buf, sem
            ).wait()

        def repack(buf, packed):
            def rows4(q, carry):
                for rr in range(4):
                    r = q * 4 + rr
                    for g in range(emb_dim // 16):
                        packed[r, pl.ds(g * 16, 16)] = buf[r, pl.ds(g * 16, 16)]
                return carry

            lax.fori_loop(0, rows_per_step // 4, rows4, 0)

        def fire_w(t, packed, wsem):
            pltpu.async_copy(
                packed,
                out_hbm.at[pl.ds(out_base + t * rows_per_step, rows_per_step)],
                wsem,
            )

        def drain_w(packed, wsem):
            pltpu.make_async_copy(
                out_hbm.at[pl.ds(0, rows_per_step)], packed, wsem
            ).wait()

        fire(0, buf0, sem0)
        fire(1, buf1, sem1)

        for b in range(2):
            drain(bufs[b], sems[b])
            repack(bufs[b], packs[b])
            fire_w(b, packs[b], wsems[b])
            fire(b + 2, bufs[b], sems[b])

        def step2(i, carry):
            tt = 2 + i * 2
            for b in range(2):
                t = tt + b
                drain(bufs[b], sems[b])
                drain_w(packs[b], wsems[b])
                repack(bufs[b], packs[b])
                fire_w(t, packs[b], wsems[b])
                fire(t + 2, bufs[b], sems[b])
            return carry

        lax.fori_loop(0, (steps - 4) // 2, step2, 0)

        for b in range(2):
            t = steps - 2 + b
            drain(bufs[b], sems[b])
            drain_w(packs[b], wsems[b])
            repack(bufs[b], packs[b])
            fire_w(t, packs[b], wsems[b])

        for b in range(2):
            drain_w(packs[b], wsems[b])

    return gather_kernel


def kernel(input_variable, embedding_weight):
    batch, hist = input_variable.shape
    vocab, emb_dim = embedding_weight.shape
    total = batch * hist
    info = plsc.get_sparse_core_info()
    n_workers = info.num_cores * info.num_subcores
    chunks = total // (n_workers * _CHUNK)

    table128 = _build_pad(vocab, emb_dim)(embedding_weight)

    idx = jnp.maximum(input_variable, 0).reshape(n_workers, chunks, _CHUNK)
    out = _build_gather(total, emb_dim, n_workers, info.num_cores)(idx, table128)
    return out.reshape(batch, hist, emb_dim)
